# Initial kernel scaffold; baseline (speedup 1.0000x reference)
import jax
import jax.numpy as jnp
from jax.experimental import pallas as pl


def kernel(x, edge_list, edge2id, edge_weight):
    raise NotImplementedError("write your pallas kernel here")


# speedup vs baseline: 7.2155x; 7.2155x over previous
import functools

import jax
import jax.numpy as jnp
from jax import lax
from jax.experimental import pallas as pl
from jax.experimental.pallas import tpu as pltpu
from jax.experimental.pallas import tpu_sc as plsc

HID = 128
NODES = 10000
EDGES = 320000
PAD = 144

NC = 2
NS = 16
NW = NC * NS
EDGES_PER_W = EDGES // NW
CHUNK = 80
STEPS = EDGES_PER_W // CHUNK
NODES_P = 10240
ROWS_PER_SUB = NODES_P // NS
ZROWS = 128


def _exp_body(w_ref, o_ref):
    w = w_ref[...]
    m = jnp.max(w)
    o_ref[...] = jnp.exp(w - m)


@jax.jit
def _prep_exp(w2d):
    return pl.pallas_call(
        _exp_body,
        out_shape=jax.ShapeDtypeStruct(w2d.shape, jnp.float32),
    )(w2d)


def _xa_body(x_ref, o_ref):
    xt = x_ref[...].T
    o_ref[:, 0:HID] = xt
    col = lax.broadcasted_iota(jnp.int32, (NODES, PAD - HID), 1)
    o_ref[:, HID:PAD] = jnp.where(col == 0, 1.0, 0.0).astype(jnp.float32)


@jax.jit
def _prep_xa(x):
    return pl.pallas_call(
        _xa_body,
        out_shape=jax.ShapeDtypeStruct((NODES, PAD), jnp.float32),
    )(x)


def _sc_body(xa_hbm, exp_hbm, src_hbm, tgt_hbm, e2i_hbm, out_hbm,
             src_v, tgt_v, e2i_v, exp_v, rows_v, zb_v, accum, sem0, sem1):
    c = lax.axis_index("c")
    s = lax.axis_index("s")
    wid = c * NS + s
    base = wid * EDGES_PER_W

    @pl.loop(0, ZROWS)
    def _(r):
        for g in range(PAD // 16):
            zb_v[r, pl.ds(16 * g, 16)] = jnp.zeros((16,), jnp.float32)

    for k in range(ROWS_PER_SUB // ZROWS):
        pltpu.sync_copy(zb_v, accum.at[pl.ds(s * ROWS_PER_SUB + k * ZROWS, ZROWS)])

    plsc.subcore_barrier()

    @pl.loop(0, STEPS)
    def _(i):
        off = base + i * CHUNK
        pltpu.sync_copy(src_hbm.at[pl.ds(off, CHUNK)], src_v)
        pltpu.sync_copy(tgt_hbm.at[pl.ds(off, CHUNK)], tgt_v)
        pltpu.sync_copy(e2i_hbm.at[pl.ds(off, CHUNK)], e2i_v)
        pltpu.async_copy(exp_hbm.at[e2i_v], exp_v, sem0).wait()
        pltpu.async_copy(xa_hbm.at[tgt_v], rows_v, sem1).wait()

        @pl.loop(0, CHUNK, step=16)
        def _(e0):
            ec = exp_v[pl.ds(e0, 16)]
            for l in range(16):
                bc = jnp.broadcast_to(ec[l], (16,))
                for g in range(PAD // 16):
                    sl = (e0 + l, pl.ds(16 * g, 16))
                    rows_v[sl] = rows_v[sl] * bc

        pltpu.sync_copy(rows_v, accum.at[src_v], add=True)

    plsc.subcore_barrier()

    for k in range(ROWS_PER_SUB // ZROWS):
        row0 = s * ROWS_PER_SUB + k * ZROWS
        pltpu.sync_copy(accum.at[pl.ds(row0, ZROWS)], zb_v)
        pltpu.sync_copy(zb_v, out_hbm.at[c, pl.ds(row0, ZROWS)])


@jax.jit
def _sc_aggregate(xa, expf, src, tgt, e2i):
    mesh = plsc.VectorSubcoreMesh(core_axis_name="c", subcore_axis_name="s")
    kern = pl.kernel(
        _sc_body,
        out_type=jax.ShapeDtypeStruct((NC, NODES_P, PAD), jnp.float32),
        mesh=mesh,
        scratch_types=[
            pltpu.VMEM((CHUNK,), jnp.int32),
            pltpu.VMEM((CHUNK,), jnp.int32),
            pltpu.VMEM((CHUNK,), jnp.int32),
            pltpu.VMEM((CHUNK,), jnp.float32),
            pltpu.VMEM((CHUNK, PAD), jnp.float32),
            pltpu.VMEM((ZROWS, PAD), jnp.float32),
            pltpu.VMEM_SHARED((NODES_P, PAD), jnp.float32),
            pltpu.SemaphoreType.DMA,
            pltpu.SemaphoreType.DMA,
        ],
        compiler_params=pltpu.CompilerParams(use_tc_tiling_on_sc=False),
    )
    return kern(xa, expf, src, tgt, e2i)


def _fin_body(p_ref, o_ref):
    p = p_ref[0] + p_ref[1]
    numer = p[0:NODES, 0:HID]
    denom = p[0:NODES, HID:HID + 1] + 1e-10
    o_ref[...] = (numer / denom).T


@jax.jit
def _finish(partial):
    return pl.pallas_call(
        _fin_body,
        out_shape=jax.ShapeDtypeStruct((HID, NODES), jnp.float32),
    )(partial)


def kernel(x, edge_list, edge2id, edge_weight):
    src = edge_list[0].astype(jnp.int32)
    tgt = edge_list[1].astype(jnp.int32)
    e2i = edge2id.astype(jnp.int32)
    w2d = edge_weight.reshape(2500, 128)

    expf = _prep_exp(w2d).reshape(EDGES)
    xa = _prep_xa(x)
    partial = _sc_aggregate(xa, expf, src, tgt, e2i)
    return _finish(partial)


# speedup vs baseline: 20.6786x; 2.8659x over previous
import jax
import jax.numpy as jnp
from jax import lax
from jax.experimental import pallas as pl
from jax.experimental.pallas import tpu as pltpu
from jax.experimental.pallas import tpu_sc as plsc

HID = 128
NODES = 10000
EDGES = 320000
PAD = 144

NC = 2
NS = 16
NW = NC * NS
EDGES_PER_W = EDGES // NW
CHUNK = 80
STEPS = EDGES_PER_W // CHUNK
NODES_P = 10240
ROWS_PER_SUB = NODES_P // NS
NCOPY = ROWS_PER_SUB // CHUNK


def _prep_body(w_ref, el_ref, x_ref, exp_ref, f_ref, st_ref, xa_ref):
    w = w_ref[...]
    m = jnp.max(w)
    mn = jnp.min(w)
    exp_ref[...] = jnp.exp(w - m)
    f_ref[...] = jnp.broadcast_to(jnp.where(m == mn, 1, 0).astype(jnp.int32), (1, 1))

    st_ref[...] = el_ref[0] * 16384 + el_ref[1]

    xa_ref[:, 0:HID] = x_ref[...].T
    col = lax.broadcasted_iota(jnp.int32, (NODES, PAD - HID), 1)
    xa_ref[:, HID:PAD] = jnp.where(col == 0, 1.0, 0.0).astype(jnp.float32)


def _prep(w2d, el3, x):
    return pl.pallas_call(
        _prep_body,
        out_shape=[
            jax.ShapeDtypeStruct((2500, 128), jnp.float32),
            jax.ShapeDtypeStruct((1, 1), jnp.int32),
            jax.ShapeDtypeStruct((2500, 128), jnp.int32),
            jax.ShapeDtypeStruct((NODES, PAD), jnp.float32),
        ],
    )(w2d, el3, x)


def _sc_body(xa_hbm, exp_hbm, st_hbm, e2i_hbm, flag_hbm, out_hbm,
             stv, flagv, rows0, rows1, src0, src1, tgt0, tgt1,
             e2ib0, e2ib1, expb0, expb1,
             sg0, sg1, se0, se1, ss0, ss1, accum):
    c = lax.axis_index("c")
    s = lax.axis_index("s")
    wid = c * NS + s
    cbase = wid * STEPS

    pltpu.sync_copy(st_hbm.at[pl.ds(cbase, STEPS)], stv)
    pltpu.sync_copy(flag_hbm, flagv)
    need_scale = flagv[...][0] == 0

    @pl.loop(0, CHUNK)
    def _(r):
        for g in range(PAD // 16):
            rows0[r, pl.ds(16 * g, 16)] = jnp.zeros((16,), jnp.float32)

    for k in range(NCOPY):
        pltpu.sync_copy(rows0, accum.at[pl.ds(s * ROWS_PER_SUB + k * CHUNK, CHUNK)])

    plsc.subcore_barrier()

    rows = (rows0, rows1)
    srcb = (src0, src1)
    tgtb = (tgt0, tgt1)
    e2ib = (e2ib0, e2ib1)
    expb = (expb0, expb1)
    sg = (sg0, sg1)
    se = (se0, se1)
    ss = (ss0, ss1)

    def issue(i, b):
        for g in range(CHUNK // 16):
            p = stv[i, pl.ds(16 * g, 16)]
            tgtb[b][pl.ds(16 * g, 16)] = p & 16383
            srcb[b][pl.ds(16 * g, 16)] = lax.shift_right_logical(p, 14)
        pltpu.async_copy(xa_hbm.at[tgtb[b]], rows[b], sg[b])

        @pl.when(need_scale)
        def _():
            pltpu.sync_copy(e2i_hbm.at[cbase + i], e2ib[b])
            pltpu.async_copy(exp_hbm.at[e2ib[b]], expb[b], se[b])

    def work(i, b):
        pltpu.make_async_copy(xa_hbm.at[tgtb[b]], rows[b], sg[b]).wait()

        @pl.when(need_scale)
        def _():
            pltpu.make_async_copy(exp_hbm.at[e2ib[b]], expb[b], se[b]).wait()

            @pl.loop(0, CHUNK, step=16)
            def _(e0):
                ec = expb[b][pl.ds(e0, 16)]
                for l in range(16):
                    bc = jnp.broadcast_to(ec[l], (16,))
                    for g in range(PAD // 16):
                        sl = (e0 + l, pl.ds(16 * g, 16))
                        rows[b][sl] = rows[b][sl] * bc

        pltpu.sync_copy(rows[b], accum.at[srcb[b]], add=True)

        @pl.when(i + 2 < STEPS)
        def _():
            issue(i + 2, b)

    issue(0, 0)
    issue(1, 1)

    @pl.loop(0, STEPS - 1, step=2)
    def _(i):
        work(i, 0)
        work(i + 1, 1)

    work(STEPS - 1, 0)

    plsc.subcore_barrier()

    for k in range(NCOPY):
        row0 = s * ROWS_PER_SUB + k * CHUNK
        pltpu.sync_copy(accum.at[pl.ds(row0, CHUNK)], rows0)
        pltpu.sync_copy(rows0, out_hbm.at[c, pl.ds(row0, CHUNK)])


def _sc_aggregate(xa, expf, st2, e2i2, flag16):
    mesh = plsc.VectorSubcoreMesh(core_axis_name="c", subcore_axis_name="s")
    kern = pl.kernel(
        _sc_body,
        out_type=jax.ShapeDtypeStruct((NC, NODES_P, PAD), jnp.float32),
        mesh=mesh,
        scratch_types=[
            pltpu.VMEM((STEPS, CHUNK), jnp.int32),
            pltpu.VMEM((16,), jnp.int32),
            pltpu.VMEM((CHUNK, PAD), jnp.float32),
            pltpu.VMEM((CHUNK, PAD), jnp.float32),
            pltpu.VMEM((CHUNK,), jnp.int32),
            pltpu.VMEM((CHUNK,), jnp.int32),
            pltpu.VMEM((CHUNK,), jnp.int32),
            pltpu.VMEM((CHUNK,), jnp.int32),
            pltpu.VMEM((CHUNK,), jnp.int32),
            pltpu.VMEM((CHUNK,), jnp.int32),
            pltpu.VMEM((CHUNK,), jnp.float32),
            pltpu.VMEM((CHUNK,), jnp.float32),
            pltpu.SemaphoreType.DMA,
            pltpu.SemaphoreType.DMA,
            pltpu.SemaphoreType.DMA,
            pltpu.SemaphoreType.DMA,
            pltpu.SemaphoreType.DMA,
            pltpu.SemaphoreType.DMA,
            pltpu.VMEM_SHARED((NODES_P, PAD), jnp.float32),
        ],
        compiler_params=pltpu.CompilerParams(use_tc_tiling_on_sc=False),
    )
    return kern(xa, expf, st2, e2i2, flag16)


def _fin_body(p_ref, o_ref):
    p = p_ref[0] + p_ref[1]
    numer = p[0:NODES, 0:HID]
    denom = p[0:NODES, HID:HID + 1] + 1e-10
    o_ref[...] = (numer / denom).T


def _finish(partial):
    return pl.pallas_call(
        _fin_body,
        out_shape=jax.ShapeDtypeStruct((HID, NODES), jnp.float32),
    )(partial)


@jax.jit
def kernel(x, edge_list, edge2id, edge_weight):
    el3 = edge_list.astype(jnp.int32).reshape(2, 2500, 128)
    e2i2 = edge2id.astype(jnp.int32).reshape(EDGES // CHUNK, CHUNK)
    w2d = edge_weight.reshape(2500, 128)

    exp2d, flag, st, xa = _prep(w2d, el3, x)
    expf = exp2d.reshape(EDGES)
    st2 = st.reshape(EDGES // CHUNK, CHUNK)
    flag16 = jnp.broadcast_to(flag[0, 0], (16,))

    partial = _sc_aggregate(xa, expf, st2, e2i2, flag16)
    return _finish(partial)


# speedup vs baseline: 20.7321x; 1.0026x over previous
import jax
import jax.numpy as jnp
from jax import lax
from jax.experimental import pallas as pl
from jax.experimental.pallas import tpu as pltpu
from jax.experimental.pallas import tpu_sc as plsc

HID = 128
NODES = 10000
EDGES = 320000
PAD = 144

NC = 2
NS = 16
NW = NC * NS
EDGES_PER_W = EDGES // NW
CHUNK = 80
STEPS = EDGES_PER_W // CHUNK
NODES_P = 10240
ROWS_PER_SUB = NODES_P // NS
NCOPY = ROWS_PER_SUB // CHUNK


def _prep_body(w_ref, el_ref, x_ref, exp_ref, f_ref, st_ref, xa_ref):
    w = w_ref[...]
    m = jnp.max(w)
    mn = jnp.min(w)
    exp_ref[...] = jnp.exp(w - m)
    f_ref[...] = jnp.broadcast_to(jnp.where(m == mn, 1, 0).astype(jnp.int32), (1, 1))

    st_ref[...] = el_ref[0] * 16384 + el_ref[1]

    xa_ref[:, 0:HID] = x_ref[...].T
    col = lax.broadcasted_iota(jnp.int32, (NODES, PAD - HID), 1)
    xa_ref[:, HID:PAD] = jnp.where(col == 0, 1.0, 0.0).astype(jnp.float32)


def _prep(w2d, el3, x):
    return pl.pallas_call(
        _prep_body,
        out_shape=[
            jax.ShapeDtypeStruct((2500, 128), jnp.float32),
            jax.ShapeDtypeStruct((1, 1), jnp.int32),
            jax.ShapeDtypeStruct((2500, 128), jnp.int32),
            jax.ShapeDtypeStruct((NODES, PAD), jnp.float32),
        ],
    )(w2d, el3, x)


def _sc_body(xa_hbm, exp_hbm, st_hbm, e2i_hbm, flag_hbm, out_hbm,
             stv, flagv, rows0, rows1, src0, src1, tgt0, tgt1,
             e2ib0, e2ib1, expb0, expb1,
             sg0, sg1, se0, se1, ss0, ss1, accum):
    c = lax.axis_index("c")
    s = lax.axis_index("s")
    wid = c * NS + s
    cbase = wid * STEPS

    pltpu.sync_copy(st_hbm.at[pl.ds(cbase, STEPS)], stv)
    pltpu.sync_copy(flag_hbm, flagv)
    need_scale = flagv[...][0] == 0

    @pl.loop(0, CHUNK)
    def _(r):
        for g in range(PAD // 16):
            rows0[r, pl.ds(16 * g, 16)] = jnp.zeros((16,), jnp.float32)

    for k in range(NCOPY):
        pltpu.sync_copy(rows0, accum.at[pl.ds(s * ROWS_PER_SUB + k * CHUNK, CHUNK)])

    plsc.subcore_barrier()

    rows = (rows0, rows1)
    srcb = (src0, src1)
    tgtb = (tgt0, tgt1)
    e2ib = (e2ib0, e2ib1)
    expb = (expb0, expb1)
    sg = (sg0, sg1)
    se = (se0, se1)
    ss = (ss0, ss1)

    def issue(i, b):
        for g in range(CHUNK // 16):
            p = stv[i, pl.ds(16 * g, 16)]
            tgtb[b][pl.ds(16 * g, 16)] = p & 16383
            srcb[b][pl.ds(16 * g, 16)] = lax.shift_right_logical(p, 14)
        pltpu.async_copy(xa_hbm.at[tgtb[b]], rows[b], sg[b])

        @pl.when(need_scale)
        def _():
            pltpu.sync_copy(e2i_hbm.at[cbase + i], e2ib[b])
            pltpu.async_copy(exp_hbm.at[e2ib[b]], expb[b], se[b])

    def work(i, b):
        pltpu.make_async_copy(xa_hbm.at[tgtb[b]], rows[b], sg[b]).wait()

        @pl.when(need_scale)
        def _():
            pltpu.make_async_copy(exp_hbm.at[e2ib[b]], expb[b], se[b]).wait()

            @pl.loop(0, CHUNK, step=16)
            def _(e0):
                ec = expb[b][pl.ds(e0, 16)]
                for l in range(16):
                    bc = jnp.broadcast_to(ec[l], (16,))
                    for g in range(PAD // 16):
                        sl = (e0 + l, pl.ds(16 * g, 16))
                        rows[b][sl] = rows[b][sl] * bc

        pltpu.sync_copy(rows[b], accum.at[srcb[b]], add=True)

        @pl.when(i + 2 < STEPS)
        def _():
            issue(i + 2, b)

    issue(0, 0)
    issue(1, 1)

    @pl.loop(0, STEPS - 1, step=2)
    def _(i):
        work(i, 0)
        work(i + 1, 1)

    work(STEPS - 1, 0)

    plsc.subcore_barrier()

    row0 = s * ROWS_PER_SUB
    pltpu.sync_copy(accum.at[pl.ds(row0, ROWS_PER_SUB)],
                    out_hbm.at[c, pl.ds(row0, ROWS_PER_SUB)])


def _sc_aggregate(xa, expf, st2, e2i2, flag16):
    mesh = plsc.VectorSubcoreMesh(core_axis_name="c", subcore_axis_name="s")
    kern = pl.kernel(
        _sc_body,
        out_type=jax.ShapeDtypeStruct((NC, NODES_P, PAD), jnp.float32),
        mesh=mesh,
        scratch_types=[
            pltpu.VMEM((STEPS, CHUNK), jnp.int32),
            pltpu.VMEM((16,), jnp.int32),
            pltpu.VMEM((CHUNK, PAD), jnp.float32),
            pltpu.VMEM((CHUNK, PAD), jnp.float32),
            pltpu.VMEM((CHUNK,), jnp.int32),
            pltpu.VMEM((CHUNK,), jnp.int32),
            pltpu.VMEM((CHUNK,), jnp.int32),
            pltpu.VMEM((CHUNK,), jnp.int32),
            pltpu.VMEM((CHUNK,), jnp.int32),
            pltpu.VMEM((CHUNK,), jnp.int32),
            pltpu.VMEM((CHUNK,), jnp.float32),
            pltpu.VMEM((CHUNK,), jnp.float32),
            pltpu.SemaphoreType.DMA,
            pltpu.SemaphoreType.DMA,
            pltpu.SemaphoreType.DMA,
            pltpu.SemaphoreType.DMA,
            pltpu.SemaphoreType.DMA,
            pltpu.SemaphoreType.DMA,
            pltpu.VMEM_SHARED((NODES_P, PAD), jnp.float32),
        ],
        compiler_params=pltpu.CompilerParams(use_tc_tiling_on_sc=False),
    )
    return kern(xa, expf, st2, e2i2, flag16)


def _fin_body(p_ref, o_ref):
    p = p_ref[0] + p_ref[1]
    numer = p[0:NODES, 0:HID]
    denom = p[0:NODES, HID:HID + 1] + 1e-10
    o_ref[...] = (numer / denom).T


def _finish(partial):
    return pl.pallas_call(
        _fin_body,
        out_shape=jax.ShapeDtypeStruct((HID, NODES), jnp.float32),
    )(partial)


@jax.jit
def kernel(x, edge_list, edge2id, edge_weight):
    el3 = edge_list.astype(jnp.int32).reshape(2, 2500, 128)
    e2i2 = edge2id.astype(jnp.int32).reshape(EDGES // CHUNK, CHUNK)
    w2d = edge_weight.reshape(2500, 128)

    exp2d, flag, st, xa = _prep(w2d, el3, x)
    expf = exp2d.reshape(EDGES)
    st2 = st.reshape(EDGES // CHUNK, CHUNK)
    flag16 = jnp.broadcast_to(flag[0, 0], (16,))

    partial = _sc_aggregate(xa, expf, st2, e2i2, flag16)
    return _finish(partial)


# speedup vs baseline: 25.4816x; 1.2291x over previous
import jax
import jax.numpy as jnp
from jax import lax
from jax.experimental import pallas as pl
from jax.experimental.pallas import tpu as pltpu
from jax.experimental.pallas import tpu_sc as plsc

HID = 128
NODES = 10000
EDGES = 320000
PAD = 128

NC = 2
NS = 16
NW = NC * NS
EDGES_PER_W = EDGES // NW
CHUNK = 80
STEPS = EDGES_PER_W // CHUNK
NODES_P = 10240
ROWS_PER_SUB = NODES_P // NS
NCOPY = ROWS_PER_SUB // CHUNK


def _prep_body(w_ref, el_ref, x_ref, exp_ref, f_ref, st_ref, xa_ref):
    w = w_ref[...]
    m = jnp.max(w)
    mn = jnp.min(w)
    exp_ref[...] = jnp.exp(w - m)
    f_ref[...] = jnp.broadcast_to(jnp.where(m == mn, 1, 0).astype(jnp.int32), (1, 1))

    st_ref[...] = el_ref[0] * 16384 + el_ref[1]

    xa_ref[...] = x_ref[...].T


def _prep(w2d, el3, x):
    return pl.pallas_call(
        _prep_body,
        out_shape=[
            jax.ShapeDtypeStruct((2500, 128), jnp.float32),
            jax.ShapeDtypeStruct((1, 1), jnp.int32),
            jax.ShapeDtypeStruct((2500, 128), jnp.int32),
            jax.ShapeDtypeStruct((NODES, PAD), jnp.float32),
        ],
    )(w2d, el3, x)


def _sc_body(xa_hbm, exp_hbm, st_hbm, e2i_hbm, flag_hbm, out_hbm, hist_hbm,
             stv, flagv, rows0, rows1, src0, src1, tgt0, tgt1,
             e2ib0, e2ib1, expb0, expb1, hist,
             sg0, sg1, se0, se1, ss0, ss1, accum):
    c = lax.axis_index("c")
    s = lax.axis_index("s")
    wid = c * NS + s
    cbase = wid * STEPS

    pltpu.sync_copy(st_hbm.at[pl.ds(cbase, STEPS)], stv)
    pltpu.sync_copy(flag_hbm, flagv)
    need_scale = flagv[...][0] == 0

    @pl.loop(0, CHUNK)
    def _(r):
        for g in range(PAD // 16):
            rows0[r, pl.ds(16 * g, 16)] = jnp.zeros((16,), jnp.float32)

    for k in range(NCOPY):
        pltpu.sync_copy(rows0, accum.at[pl.ds(s * ROWS_PER_SUB + k * CHUNK, CHUNK)])

    @pl.loop(0, NODES_P, step=16)
    def _(r):
        hist[pl.ds(r, 16)] = jnp.zeros((16,), jnp.float32)

    plsc.subcore_barrier()

    rows = (rows0, rows1)
    srcb = (src0, src1)
    tgtb = (tgt0, tgt1)
    e2ib = (e2ib0, e2ib1)
    expb = (expb0, expb1)
    sg = (sg0, sg1)
    se = (se0, se1)
    ss = (ss0, ss1)

    def issue(i, b):
        for g in range(CHUNK // 16):
            p = stv[i, pl.ds(16 * g, 16)]
            tgtb[b][pl.ds(16 * g, 16)] = p & 16383
            srcb[b][pl.ds(16 * g, 16)] = lax.shift_right_logical(p, 14)
        pltpu.async_copy(xa_hbm.at[tgtb[b]], rows[b], sg[b])

        @pl.when(need_scale)
        def _():
            pltpu.sync_copy(e2i_hbm.at[cbase + i], e2ib[b])
            pltpu.async_copy(exp_hbm.at[e2ib[b]], expb[b], se[b])

    def work(i, b):
        pltpu.make_async_copy(xa_hbm.at[tgtb[b]], rows[b], sg[b]).wait()

        @pl.when(need_scale)
        def _():
            pltpu.make_async_copy(exp_hbm.at[e2ib[b]], expb[b], se[b]).wait()

            @pl.loop(0, CHUNK, step=16)
            def _(e0):
                ec = expb[b][pl.ds(e0, 16)]
                idx = srcb[b][pl.ds(e0, 16)]
                plsc.addupdate_scatter(hist, [idx], ec)
                for l in range(16):
                    bc = jnp.broadcast_to(ec[l], (16,))
                    for g in range(PAD // 16):
                        sl = (e0 + l, pl.ds(16 * g, 16))
                        rows[b][sl] = rows[b][sl] * bc

        @pl.when(jnp.logical_not(need_scale))
        def _():
            ones = jnp.ones((16,), jnp.float32)
            for g in range(CHUNK // 16):
                idx = srcb[b][pl.ds(16 * g, 16)]
                plsc.addupdate_scatter(hist, [idx], ones)

        pltpu.sync_copy(rows[b], accum.at[srcb[b]], add=True)

        @pl.when(i + 2 < STEPS)
        def _():
            issue(i + 2, b)

    issue(0, 0)
    issue(1, 1)

    @pl.loop(0, STEPS - 1, step=2)
    def _(i):
        work(i, 0)
        work(i + 1, 1)

    work(STEPS - 1, 0)

    plsc.subcore_barrier()

    row0 = s * ROWS_PER_SUB
    pltpu.sync_copy(accum.at[pl.ds(row0, ROWS_PER_SUB)],
                    out_hbm.at[c, pl.ds(row0, ROWS_PER_SUB)])
    pltpu.sync_copy(hist, hist_hbm.at[c, s])


def _sc_aggregate(xa, expf, st2, e2i2, flag16):
    mesh = plsc.VectorSubcoreMesh(core_axis_name="c", subcore_axis_name="s")
    kern = pl.kernel(
        _sc_body,
        out_type=[
            jax.ShapeDtypeStruct((NC, NODES_P, PAD), jnp.float32),
            jax.ShapeDtypeStruct((NC, NS, NODES_P), jnp.float32),
        ],
        mesh=mesh,
        scratch_types=[
            pltpu.VMEM((STEPS, CHUNK), jnp.int32),
            pltpu.VMEM((16,), jnp.int32),
            pltpu.VMEM((CHUNK, PAD), jnp.float32),
            pltpu.VMEM((CHUNK, PAD), jnp.float32),
            pltpu.VMEM((CHUNK,), jnp.int32),
            pltpu.VMEM((CHUNK,), jnp.int32),
            pltpu.VMEM((CHUNK,), jnp.int32),
            pltpu.VMEM((CHUNK,), jnp.int32),
            pltpu.VMEM((CHUNK,), jnp.int32),
            pltpu.VMEM((CHUNK,), jnp.int32),
            pltpu.VMEM((CHUNK,), jnp.float32),
            pltpu.VMEM((CHUNK,), jnp.float32),
            pltpu.VMEM((NODES_P,), jnp.float32),
            pltpu.SemaphoreType.DMA,
            pltpu.SemaphoreType.DMA,
            pltpu.SemaphoreType.DMA,
            pltpu.SemaphoreType.DMA,
            pltpu.SemaphoreType.DMA,
            pltpu.SemaphoreType.DMA,
            pltpu.VMEM_SHARED((NODES_P, PAD), jnp.float32),
        ],
        compiler_params=pltpu.CompilerParams(
            use_tc_tiling_on_sc=False, needs_layout_passes=False),
    )
    return kern(xa, expf, st2, e2i2, flag16)


def _fin_body(p_ref, h_ref, o_ref):
    p = p_ref[0] + p_ref[1]
    numer = p[0:NODES, :].T
    hs = jnp.sum(h_ref[...], axis=0)
    denom = hs[0:NODES][None, :] + 1e-10
    o_ref[...] = numer / denom


def _finish(partial, hists):
    return pl.pallas_call(
        _fin_body,
        out_shape=jax.ShapeDtypeStruct((HID, NODES), jnp.float32),
    )(partial, hists)


@jax.jit
def kernel(x, edge_list, edge2id, edge_weight):
    el3 = edge_list.astype(jnp.int32).reshape(2, 2500, 128)
    e2i2 = edge2id.astype(jnp.int32).reshape(EDGES // CHUNK, CHUNK)
    w2d = edge_weight.reshape(2500, 128)

    exp2d, flag, st, xa = _prep(w2d, el3, x)
    expf = exp2d.reshape(EDGES)
    st2 = st.reshape(EDGES // CHUNK, CHUNK)
    flag16 = jnp.broadcast_to(flag[0, 0], (16,))

    partial, hists = _sc_aggregate(xa, expf, st2, e2i2, flag16)
    return _finish(partial, hists.reshape(NC * NS, NODES_P))


# speedup vs baseline: 29.6145x; 1.1622x over previous
import jax
import jax.numpy as jnp
from jax import lax
from jax.experimental import pallas as pl
from jax.experimental.pallas import tpu as pltpu
from jax.experimental.pallas import tpu_sc as plsc

HID = 128
NODES = 10000
EDGES = 320000
PAD = 128

NC = 2
NS = 16
NW = NC * NS
EDGES_PER_W = EDGES // NW
CHUNK = 80
STEPS = EDGES_PER_W // CHUNK
NODES_P = 10240
ROWS_PER_SUB = NODES_P // NS
NCOPY = ROWS_PER_SUB // CHUNK


def _prep_body(w_ref, el_ref, x_ref, exp_ref, f_ref, st_ref, xa_ref):
    w = w_ref[...]
    m = jnp.max(w)
    mn = jnp.min(w)
    exp_ref[...] = jnp.exp(w - m)
    f_ref[...] = jnp.broadcast_to(jnp.where(m == mn, 1, 0).astype(jnp.int32), (1, 1))

    st_ref[...] = el_ref[0] * 16384 + el_ref[1]

    xa_ref[...] = x_ref[...].T


def _prep(w2d, el3, x):
    return pl.pallas_call(
        _prep_body,
        out_shape=[
            jax.ShapeDtypeStruct((2500, 128), jnp.float32),
            jax.ShapeDtypeStruct((1, 1), jnp.int32),
            jax.ShapeDtypeStruct((2500, 128), jnp.int32),
            jax.ShapeDtypeStruct((NODES, PAD), jnp.float32),
        ],
    )(w2d, el3, x)


def _sc_body(xa_hbm, exp_hbm, st_hbm, e2i_hbm, flag_hbm, out_hbm, hist_hbm,
             flagv, rows0, rows1, rows2, stb0, stb1, stb2,
             src0, src1, src2, tgt0, tgt1, tgt2,
             e2ib0, e2ib1, e2ib2, expb0, expb1, expb2, hist,
             si0, si1, si2, sg0, sg1, sg2, se0, se1, se2, accum):
    c = lax.axis_index("c")
    s = lax.axis_index("s")
    wid = c * NS + s
    ebase = wid * EDGES_PER_W

    pltpu.sync_copy(flag_hbm, flagv)
    need_scale = flagv[...][0] == 0

    @pl.loop(0, CHUNK)
    def _(r):
        for g in range(PAD // 16):
            rows0[r, pl.ds(16 * g, 16)] = jnp.zeros((16,), jnp.float32)

    for k in range(NCOPY):
        pltpu.sync_copy(rows0, accum.at[pl.ds(s * ROWS_PER_SUB + k * CHUNK, CHUNK)])

    @pl.loop(0, NODES_P, step=16)
    def _(r):
        hist[pl.ds(r, 16)] = jnp.zeros((16,), jnp.float32)

    plsc.subcore_barrier()

    rows = (rows0, rows1, rows2)
    stb = (stb0, stb1, stb2)
    srcb = (src0, src1, src2)
    tgtb = (tgt0, tgt1, tgt2)
    e2ib = (e2ib0, e2ib1, e2ib2)
    expb = (expb0, expb1, expb2)
    si = (si0, si1, si2)
    sg = (sg0, sg1, sg2)
    se = (se0, se1, se2)

    def prefetch_idx(i, b):
        pltpu.async_copy(st_hbm.at[pl.ds(ebase + i * CHUNK, CHUNK)], stb[b], si[b])

    def issue(i, b):
        pltpu.make_async_copy(st_hbm.at[pl.ds(ebase, CHUNK)], stb[b], si[b]).wait()
        for g in range(CHUNK // 16):
            p = stb[b][pl.ds(16 * g, 16)]
            tgtb[b][pl.ds(16 * g, 16)] = p & 16383
            srcb[b][pl.ds(16 * g, 16)] = lax.shift_right_logical(p, 14)
        pltpu.async_copy(xa_hbm.at[tgtb[b]], rows[b], sg[b])

        @pl.when(need_scale)
        def _():
            pltpu.sync_copy(e2i_hbm.at[pl.ds(ebase + i * CHUNK, CHUNK)], e2ib[b])
            pltpu.async_copy(exp_hbm.at[e2ib[b]], expb[b], se[b])

        @pl.when(i + 3 < STEPS)
        def _():
            prefetch_idx(i + 3, b)

    def work(i, b):
        pltpu.make_async_copy(xa_hbm.at[tgtb[b]], rows[b], sg[b]).wait()

        @pl.when(need_scale)
        def _():
            pltpu.make_async_copy(exp_hbm.at[e2ib[b]], expb[b], se[b]).wait()

            @pl.loop(0, CHUNK, step=16)
            def _(e0):
                ec = expb[b][pl.ds(e0, 16)]
                idx = srcb[b][pl.ds(e0, 16)]
                plsc.addupdate_scatter(hist, [idx], ec)
                for l in range(16):
                    bc = jnp.broadcast_to(ec[l], (16,))
                    for g in range(PAD // 16):
                        sl = (e0 + l, pl.ds(16 * g, 16))
                        rows[b][sl] = rows[b][sl] * bc

        @pl.when(jnp.logical_not(need_scale))
        def _():
            ones = jnp.ones((16,), jnp.float32)
            for g in range(CHUNK // 16):
                idx = srcb[b][pl.ds(16 * g, 16)]
                plsc.addupdate_scatter(hist, [idx], ones)

        pltpu.sync_copy(rows[b], accum.at[srcb[b]], add=True)

        @pl.when(i + 3 < STEPS)
        def _():
            issue(i + 3, b)

    prefetch_idx(0, 0)
    prefetch_idx(1, 1)
    prefetch_idx(2, 2)
    issue(0, 0)
    issue(1, 1)
    issue(2, 2)

    @pl.loop(0, STEPS - 2, step=3)
    def _(i):
        work(i, 0)
        work(i + 1, 1)
        work(i + 2, 2)

    work(STEPS - 2, 0)
    work(STEPS - 1, 1)

    plsc.subcore_barrier()

    row0 = s * ROWS_PER_SUB
    pltpu.sync_copy(accum.at[pl.ds(row0, ROWS_PER_SUB)],
                    out_hbm.at[c, pl.ds(row0, ROWS_PER_SUB)])
    pltpu.sync_copy(hist, hist_hbm.at[c, s])


def _sc_aggregate(xa, expf, st2, e2i2, flag16):
    mesh = plsc.VectorSubcoreMesh(core_axis_name="c", subcore_axis_name="s")
    kern = pl.kernel(
        _sc_body,
        out_type=[
            jax.ShapeDtypeStruct((NC, NODES_P, PAD), jnp.float32),
            jax.ShapeDtypeStruct((NC, NS, NODES_P), jnp.float32),
        ],
        mesh=mesh,
        scratch_types=[
            pltpu.VMEM((16,), jnp.int32),
            pltpu.VMEM((CHUNK, PAD), jnp.float32),
            pltpu.VMEM((CHUNK, PAD), jnp.float32),
            pltpu.VMEM((CHUNK, PAD), jnp.float32),
            pltpu.VMEM((CHUNK,), jnp.int32),
            pltpu.VMEM((CHUNK,), jnp.int32),
            pltpu.VMEM((CHUNK,), jnp.int32),
            pltpu.VMEM((CHUNK,), jnp.int32),
            pltpu.VMEM((CHUNK,), jnp.int32),
            pltpu.VMEM((CHUNK,), jnp.int32),
            pltpu.VMEM((CHUNK,), jnp.int32),
            pltpu.VMEM((CHUNK,), jnp.int32),
            pltpu.VMEM((CHUNK,), jnp.int32),
            pltpu.VMEM((CHUNK,), jnp.int32),
            pltpu.VMEM((CHUNK,), jnp.int32),
            pltpu.VMEM((CHUNK,), jnp.int32),
            pltpu.VMEM((CHUNK,), jnp.float32),
            pltpu.VMEM((CHUNK,), jnp.float32),
            pltpu.VMEM((CHUNK,), jnp.float32),
            pltpu.VMEM((NODES_P,), jnp.float32),
            pltpu.SemaphoreType.DMA,
            pltpu.SemaphoreType.DMA,
            pltpu.SemaphoreType.DMA,
            pltpu.SemaphoreType.DMA,
            pltpu.SemaphoreType.DMA,
            pltpu.SemaphoreType.DMA,
            pltpu.SemaphoreType.DMA,
            pltpu.SemaphoreType.DMA,
            pltpu.SemaphoreType.DMA,
            pltpu.VMEM_SHARED((NODES_P, PAD), jnp.float32),
        ],
        compiler_params=pltpu.CompilerParams(
            use_tc_tiling_on_sc=False, needs_layout_passes=False),
    )
    return kern(xa, expf, st2, e2i2, flag16)


def _fin_body(p_ref, h_ref, o_ref):
    p = p_ref[0] + p_ref[1]
    numer = p[0:NODES, :].T
    hs = jnp.sum(h_ref[...], axis=0)
    denom = hs[0:NODES][None, :] + 1e-10
    o_ref[...] = numer / denom


def _finish(partial, hists):
    return pl.pallas_call(
        _fin_body,
        out_shape=jax.ShapeDtypeStruct((HID, NODES), jnp.float32),
    )(partial, hists)


@jax.jit
def kernel(x, edge_list, edge2id, edge_weight):
    el3 = edge_list.astype(jnp.int32).reshape(2, 2500, 128)
    e2i1 = edge2id.astype(jnp.int32)
    w2d = edge_weight.reshape(2500, 128)

    exp2d, flag, st, xa = _prep(w2d, el3, x)
    expf = exp2d.reshape(EDGES)
    st1 = st.reshape(EDGES)
    flag16 = jnp.broadcast_to(flag[0, 0], (16,))

    partial, hists = _sc_aggregate(xa, expf, st1, e2i1, flag16)
    return _finish(partial, hists.reshape(NC * NS, NODES_P))
